# baseline (speedup 1.0000x reference)
import functools

import jax
import jax.numpy as jnp
from jax import lax
from jax.experimental import pallas as pl
from jax.experimental.pallas import tpu as pltpu
from jax.experimental.pallas import tpu_sc as plsc

N_NODES = 10000
N_PAD = 10240
N_EDGES = 320000
F = 128
NW = 32
CH = 32
EPW = N_EDGES // NW
NCH = 320
SUP = 16
NBUF = 8
ROWS_PER_SUB = N_PAD // 16

_MESH = plsc.VectorSubcoreMesh(core_axis_name="c", subcore_axis_name="s")


NGRP = NCH // SUP


@functools.partial(
    pl.kernel,
    out_type=jax.ShapeDtypeStruct((2, N_PAD, F), jnp.float32),
    mesh=_MESH,
    scratch_types=[
        pltpu.VMEM((SUP, CH), jnp.int32),
        pltpu.VMEM((SUP, CH), jnp.int32),
        pltpu.VMEM((SUP, CH), jnp.int32),
        pltpu.VMEM((SUP, CH), jnp.int32),
        pltpu.VMEM((CH, F), jnp.float32),
        pltpu.VMEM((CH, F), jnp.float32),
        pltpu.VMEM((CH, F), jnp.float32),
        pltpu.VMEM((CH, F), jnp.float32),
        pltpu.VMEM((CH, F), jnp.float32),
        pltpu.VMEM((CH, F), jnp.float32),
        pltpu.VMEM((CH, F), jnp.float32),
        pltpu.VMEM((CH, F), jnp.float32),
        pltpu.VMEM_SHARED((N_PAD, F), jnp.float32),
        pltpu.SemaphoreType.DMA,
        pltpu.SemaphoreType.DMA,
        pltpu.SemaphoreType.DMA,
        pltpu.SemaphoreType.DMA,
        pltpu.SemaphoreType.DMA,
        pltpu.SemaphoreType.DMA,
        pltpu.SemaphoreType.DMA,
        pltpu.SemaphoreType.DMA,
        pltpu.SemaphoreType.DMA,
    ],
)
def _agg(x_hbm, src_hbm, dst_hbm, out_hbm, srcA, dstA, srcB, dstB,
         buf0, buf1, buf2, buf3, buf4, buf5, buf6, buf7, acc_sh,
         gsem0, gsem1, gsem2, gsem3, gsem4, gsem5, gsem6, gsem7, isem):
    bufs = (buf0, buf1, buf2, buf3, buf4, buf5, buf6, buf7)
    gsems = (gsem0, gsem1, gsem2, gsem3, gsem4, gsem5, gsem6, gsem7)
    idxs = ((srcA, dstA), (srcB, dstB))
    cid = lax.axis_index("c")
    sid = lax.axis_index("s")
    wid = sid * 2 + cid

    @pl.loop(0, CH)
    def _(i):
        @pl.loop(0, F // 16)
        def _(k):
            buf0[i, pl.ds(k * 16, 16)] = jnp.zeros((16,), jnp.float32)

    @pl.loop(0, ROWS_PER_SUB // CH)
    def _(r):
        pltpu.sync_copy(buf0, acc_sh.at[pl.ds(sid * ROWS_PER_SUB + r * CH, CH)])

    plsc.subcore_barrier()

    def wait_gather(b):
        pltpu.make_async_copy(x_hbm.at[srcA.at[0]], bufs[b], gsems[b]).wait()

    def stage(s, par):
        base = wid * NCH + s * SUP
        pltpu.async_copy(src_hbm.at[pl.ds(base, SUP)], idxs[par][0], isem)
        pltpu.async_copy(dst_hbm.at[pl.ds(base, SUP)], idxs[par][1], isem)

    def wait_stage():
        pltpu.make_async_copy(src_hbm.at[pl.ds(0, SUP)], srcA, isem).wait()
        pltpu.make_async_copy(src_hbm.at[pl.ds(0, SUP)], srcB, isem).wait()

    stage(0, 0)
    wait_stage()
    for b in range(NBUF):
        pltpu.async_copy(x_hbm.at[srcA.at[b]], bufs[b], gsems[b])

    @pl.loop(0, NGRP // 2)
    def _(g2):
        for par in range(2):
            s = g2 * 2 + par
            cur_s, cur_d = idxs[par]
            nxt_s, _ = idxs[1 - par]

            @pl.when(s + 1 < NGRP)
            def _():
                stage(s + 1, 1 - par)

            for u in range(SUP):
                b = u % NBUF
                wait_gather(b)
                pltpu.sync_copy(bufs[b], acc_sh.at[cur_d.at[u]], add=True)
                if u == SUP - NBUF:
                    @pl.when(s + 1 < NGRP)
                    def _():
                        wait_stage()
                nxt_row = (cur_s.at[u + NBUF] if u + NBUF < SUP
                           else nxt_s.at[u + NBUF - SUP])

                @pl.when(s * SUP + u + NBUF < NCH)
                def _():
                    pltpu.async_copy(x_hbm.at[nxt_row], bufs[b], gsems[b])

    plsc.subcore_barrier()
    pltpu.sync_copy(
        acc_sh.at[pl.ds(sid * ROWS_PER_SUB, ROWS_PER_SUB)],
        out_hbm.at[cid, pl.ds(sid * ROWS_PER_SUB, ROWS_PER_SUB)],
    )


DCH = 128


@functools.partial(
    pl.kernel,
    out_type=jax.ShapeDtypeStruct((2, N_PAD, F), jnp.float32),
    mesh=_MESH,
    scratch_types=[
        pltpu.VMEM((NCH * CH // DCH, DCH), jnp.int32),
        pltpu.VMEM((DCH, F), jnp.float32),
        pltpu.VMEM_SHARED((N_PAD, F), jnp.float32),
        pltpu.SemaphoreType.DMA,
        pltpu.SemaphoreType.DMA,
    ],
)
def _deg_kernel(dst_hbm, out_hbm, dst_v, ones_v, acc_sh, ssem0, ssem1):
    ssems = (ssem0, ssem1)
    KCH = NCH * CH // DCH
    cid = lax.axis_index("c")
    sid = lax.axis_index("s")
    wid = sid * 2 + cid

    @pl.loop(0, DCH)
    def _(i):
        @pl.loop(0, F // 16)
        def _(k):
            ones_v[i, pl.ds(k * 16, 16)] = jnp.zeros((16,), jnp.float32)

    @pl.loop(0, ROWS_PER_SUB // DCH)
    def _(r):
        pltpu.sync_copy(ones_v, acc_sh.at[pl.ds(sid * ROWS_PER_SUB + r * DCH, DCH)])

    @pl.loop(0, DCH)
    def _(i):
        @pl.loop(0, F // 16)
        def _(k):
            ones_v[i, pl.ds(k * 16, 16)] = jnp.ones((16,), jnp.float32)

    plsc.subcore_barrier()
    pltpu.sync_copy(dst_hbm.at[pl.ds(wid * KCH, KCH)], dst_v)

    pltpu.async_copy(ones_v, acc_sh.at[dst_v.at[0]], ssem0, add=True)
    pltpu.async_copy(ones_v, acc_sh.at[dst_v.at[1]], ssem1, add=True)

    @pl.loop(0, KCH // 2)
    def _(t):
        for par in range(2):
            pltpu.make_async_copy(
                ones_v, acc_sh.at[dst_v.at[0]], ssems[par]).wait()

            @pl.when(t * 2 + par + 2 < KCH)
            def _():
                pltpu.async_copy(
                    ones_v, acc_sh.at[dst_v.at[t * 2 + par + 2]],
                    ssems[par], add=True)

    plsc.subcore_barrier()
    pltpu.sync_copy(
        acc_sh.at[pl.ds(sid * ROWS_PER_SUB, ROWS_PER_SUB)],
        out_hbm.at[cid, pl.ds(sid * ROWS_PER_SUB, ROWS_PER_SUB)],
    )


_BS = 2048


def _combine(x, w_self, b, w_neigh, aggp, degp):
    D = w_self.shape[0]

    def body(x_ref, ws_ref, b_ref, wn_ref, a_ref, d_ref, o_ref):
        z = lax.dot_general(
            x_ref[...], ws_ref[...], (((1,), (1,)), ((), ())),
            preferred_element_type=jnp.float32,
        )
        deg = d_ref[0, :, 0:1] + d_ref[1, :, 0:1]
        inv = 1.0 / jnp.maximum(deg, 1.0)
        mean = (a_ref[0] + a_ref[1]) * inv
        mn = lax.dot_general(
            mean, wn_ref[...], (((1,), (1,)), ((), ())),
            preferred_element_type=jnp.float32,
        )
        o_ref[...] = jax.nn.sigmoid(z + b_ref[...] + mn)

    return pl.pallas_call(
        body,
        grid=(N_PAD // _BS,),
        in_specs=[
            pl.BlockSpec((_BS, F), lambda i: (i, 0)),
            pl.BlockSpec((D, F), lambda i: (0, 0)),
            pl.BlockSpec((1, D), lambda i: (0, 0)),
            pl.BlockSpec((D, F), lambda i: (0, 0)),
            pl.BlockSpec((2, _BS, F), lambda i: (0, i, 0)),
            pl.BlockSpec((2, _BS, F), lambda i: (0, i, 0)),
        ],
        out_specs=pl.BlockSpec((_BS, D), lambda i: (i, 0)),
        out_shape=jax.ShapeDtypeStruct((N_PAD, D), jnp.float32),
    )(x, w_self, b.reshape(1, D), w_neigh, aggp, degp)


def kernel(inputs, W1_self, W1_neigh, b1, W2_self, W2_neigh, b2,
           W3_self, W3_neigh, b3, edge_index):
    x = jnp.pad(inputs, ((0, N_PAD - N_NODES), (0, 0)))
    pad2 = ((0, 0), (0, NCH * CH - EPW))
    srcm = jnp.pad(edge_index[0].reshape(NW, EPW), pad2,
                   constant_values=N_NODES).reshape(NW * NCH, CH)
    dstm = jnp.pad(edge_index[1].reshape(NW, EPW), pad2,
                   constant_values=N_NODES).reshape(NW * NCH, CH)

    degp = _deg_kernel(dstm.reshape(-1, 128))

    h = x
    for w_self, w_neigh, b in ((W1_self, W1_neigh, b1),
                               (W2_self, W2_neigh, b2),
                               (W3_self, W3_neigh, b3)):
        aggp = _agg(h, srcm, dstm)
        h = _combine(h, w_self, b, w_neigh, aggp, degp)

    return h[:N_NODES]


# speedup vs baseline: 1.0056x; 1.0056x over previous
import functools

import jax
import jax.numpy as jnp
from jax import lax
from jax.experimental import pallas as pl
from jax.experimental.pallas import tpu as pltpu
from jax.experimental.pallas import tpu_sc as plsc

N_NODES = 10000
N_PAD = 10240
N_EDGES = 320000
F = 128
NW = 32
CH = 64
EPW = N_EDGES // NW
NCH = 160
SUP = 8
NBUF = 4
ROWS_PER_SUB = N_PAD // 16

_MESH = plsc.VectorSubcoreMesh(core_axis_name="c", subcore_axis_name="s")


NGRP = NCH // SUP


@functools.partial(
    pl.kernel,
    out_type=jax.ShapeDtypeStruct((2, N_PAD, F), jnp.float32),
    mesh=_MESH,
    scratch_types=[
        pltpu.VMEM((SUP, CH), jnp.int32),
        pltpu.VMEM((SUP, CH), jnp.int32),
        pltpu.VMEM((SUP, CH), jnp.int32),
        pltpu.VMEM((SUP, CH), jnp.int32),
        pltpu.VMEM((CH, F), jnp.float32),
        pltpu.VMEM((CH, F), jnp.float32),
        pltpu.VMEM((CH, F), jnp.float32),
        pltpu.VMEM((CH, F), jnp.float32),
        pltpu.VMEM_SHARED((N_PAD, F), jnp.float32),
        pltpu.SemaphoreType.DMA,
        pltpu.SemaphoreType.DMA,
        pltpu.SemaphoreType.DMA,
        pltpu.SemaphoreType.DMA,
        pltpu.SemaphoreType.DMA,
    ],
)
def _agg(x_hbm, src_hbm, dst_hbm, out_hbm, srcA, dstA, srcB, dstB,
         buf0, buf1, buf2, buf3, acc_sh, gsem0, gsem1, gsem2, gsem3, isem):
    bufs = (buf0, buf1, buf2, buf3)
    gsems = (gsem0, gsem1, gsem2, gsem3)
    idxs = ((srcA, dstA), (srcB, dstB))
    cid = lax.axis_index("c")
    sid = lax.axis_index("s")
    wid = sid * 2 + cid

    @pl.loop(0, CH)
    def _(i):
        @pl.loop(0, F // 16)
        def _(k):
            buf0[i, pl.ds(k * 16, 16)] = jnp.zeros((16,), jnp.float32)

    @pl.loop(0, ROWS_PER_SUB // CH)
    def _(r):
        pltpu.sync_copy(buf0, acc_sh.at[pl.ds(sid * ROWS_PER_SUB + r * CH, CH)])

    plsc.subcore_barrier()

    def wait_gather(b):
        pltpu.make_async_copy(x_hbm.at[srcA.at[0]], bufs[b], gsems[b]).wait()

    def stage(s, par):
        base = wid * NCH + s * SUP
        pltpu.async_copy(src_hbm.at[pl.ds(base, SUP)], idxs[par][0], isem)
        pltpu.async_copy(dst_hbm.at[pl.ds(base, SUP)], idxs[par][1], isem)

    def wait_stage():
        pltpu.make_async_copy(src_hbm.at[pl.ds(0, SUP)], srcA, isem).wait()
        pltpu.make_async_copy(src_hbm.at[pl.ds(0, SUP)], srcB, isem).wait()

    stage(0, 0)
    wait_stage()
    for b in range(NBUF):
        pltpu.async_copy(x_hbm.at[srcA.at[b]], bufs[b], gsems[b])

    @pl.loop(0, NGRP // 2)
    def _(g2):
        for par in range(2):
            s = g2 * 2 + par
            cur_s, cur_d = idxs[par]
            nxt_s, _ = idxs[1 - par]

            @pl.when(s + 1 < NGRP)
            def _():
                stage(s + 1, 1 - par)

            for u in range(SUP):
                b = u % NBUF
                wait_gather(b)
                pltpu.sync_copy(bufs[b], acc_sh.at[cur_d.at[u]], add=True)
                if u == SUP - NBUF:
                    @pl.when(s + 1 < NGRP)
                    def _():
                        wait_stage()
                nxt_row = (cur_s.at[u + NBUF] if u + NBUF < SUP
                           else nxt_s.at[u + NBUF - SUP])

                @pl.when(s * SUP + u + NBUF < NCH)
                def _():
                    pltpu.async_copy(x_hbm.at[nxt_row], bufs[b], gsems[b])

    plsc.subcore_barrier()
    pltpu.sync_copy(
        acc_sh.at[pl.ds(sid * ROWS_PER_SUB, ROWS_PER_SUB)],
        out_hbm.at[cid, pl.ds(sid * ROWS_PER_SUB, ROWS_PER_SUB)],
    )


DCH = 128


@functools.partial(
    pl.kernel,
    out_type=jax.ShapeDtypeStruct((2, N_PAD, F), jnp.float32),
    mesh=_MESH,
    scratch_types=[
        pltpu.VMEM((NCH * CH // DCH, DCH), jnp.int32),
        pltpu.VMEM((DCH, F), jnp.float32),
        pltpu.VMEM_SHARED((N_PAD, F), jnp.float32),
        pltpu.SemaphoreType.DMA,
        pltpu.SemaphoreType.DMA,
    ],
)
def _deg_kernel(dst_hbm, out_hbm, dst_v, ones_v, acc_sh, ssem0, ssem1):
    ssems = (ssem0, ssem1)
    KCH = NCH * CH // DCH
    cid = lax.axis_index("c")
    sid = lax.axis_index("s")
    wid = sid * 2 + cid

    @pl.loop(0, DCH)
    def _(i):
        @pl.loop(0, F // 16)
        def _(k):
            ones_v[i, pl.ds(k * 16, 16)] = jnp.zeros((16,), jnp.float32)

    @pl.loop(0, ROWS_PER_SUB // DCH)
    def _(r):
        pltpu.sync_copy(ones_v, acc_sh.at[pl.ds(sid * ROWS_PER_SUB + r * DCH, DCH)])

    @pl.loop(0, DCH)
    def _(i):
        @pl.loop(0, F // 16)
        def _(k):
            ones_v[i, pl.ds(k * 16, 16)] = jnp.ones((16,), jnp.float32)

    plsc.subcore_barrier()
    pltpu.sync_copy(dst_hbm.at[pl.ds(wid * KCH, KCH)], dst_v)

    pltpu.async_copy(ones_v, acc_sh.at[dst_v.at[0]], ssem0, add=True)
    pltpu.async_copy(ones_v, acc_sh.at[dst_v.at[1]], ssem1, add=True)

    @pl.loop(0, KCH // 2)
    def _(t):
        for par in range(2):
            pltpu.make_async_copy(
                ones_v, acc_sh.at[dst_v.at[0]], ssems[par]).wait()

            @pl.when(t * 2 + par + 2 < KCH)
            def _():
                pltpu.async_copy(
                    ones_v, acc_sh.at[dst_v.at[t * 2 + par + 2]],
                    ssems[par], add=True)

    plsc.subcore_barrier()
    pltpu.sync_copy(
        acc_sh.at[pl.ds(sid * ROWS_PER_SUB, ROWS_PER_SUB)],
        out_hbm.at[cid, pl.ds(sid * ROWS_PER_SUB, ROWS_PER_SUB)],
    )


_BS = 2048


def _combine(x, w_self, b, w_neigh, aggp, degp):
    D = w_self.shape[0]

    def body(x_ref, ws_ref, b_ref, wn_ref, a_ref, d_ref, o_ref):
        z = lax.dot_general(
            x_ref[...], ws_ref[...], (((1,), (1,)), ((), ())),
            preferred_element_type=jnp.float32,
        )
        deg = d_ref[0, :, 0:1] + d_ref[1, :, 0:1]
        inv = 1.0 / jnp.maximum(deg, 1.0)
        mean = (a_ref[0] + a_ref[1]) * inv
        mn = lax.dot_general(
            mean, wn_ref[...], (((1,), (1,)), ((), ())),
            preferred_element_type=jnp.float32,
        )
        o_ref[...] = jax.nn.sigmoid(z + b_ref[...] + mn)

    return pl.pallas_call(
        body,
        grid=(N_PAD // _BS,),
        in_specs=[
            pl.BlockSpec((_BS, F), lambda i: (i, 0)),
            pl.BlockSpec((D, F), lambda i: (0, 0)),
            pl.BlockSpec((1, D), lambda i: (0, 0)),
            pl.BlockSpec((D, F), lambda i: (0, 0)),
            pl.BlockSpec((2, _BS, F), lambda i: (0, i, 0)),
            pl.BlockSpec((2, _BS, F), lambda i: (0, i, 0)),
        ],
        out_specs=pl.BlockSpec((_BS, D), lambda i: (i, 0)),
        out_shape=jax.ShapeDtypeStruct((N_PAD, D), jnp.float32),
    )(x, w_self, b.reshape(1, D), w_neigh, aggp, degp)


def kernel(inputs, W1_self, W1_neigh, b1, W2_self, W2_neigh, b2,
           W3_self, W3_neigh, b3, edge_index):
    x = jnp.pad(inputs, ((0, N_PAD - N_NODES), (0, 0)))
    pad2 = ((0, 0), (0, NCH * CH - EPW))
    srcm = jnp.pad(edge_index[0].reshape(NW, EPW), pad2,
                   constant_values=N_NODES).reshape(NW * NCH, CH)
    dstm = jnp.pad(edge_index[1].reshape(NW, EPW), pad2,
                   constant_values=N_NODES).reshape(NW * NCH, CH)

    degp = _deg_kernel(dstm.reshape(-1, 128))

    h = x
    for w_self, w_neigh, b in ((W1_self, W1_neigh, b1),
                               (W2_self, W2_neigh, b2),
                               (W3_self, W3_neigh, b3)):
        aggp = _agg(h, srcm, dstm)
        h = _combine(h, w_self, b, w_neigh, aggp, degp)

    return h[:N_NODES]
